# baseline (speedup 1.0000x reference)
import functools

import jax
import jax.numpy as jnp
from jax import lax
from jax.experimental import pallas as pl
from jax.experimental.pallas import tpu as pltpu
from jax.experimental.pallas import tpu_sc as plsc

NUM_CLASSES = 1000000
FEAT_DIM = 32
BATCH = 16384

NC = 2
NS = 16
NW = NC * NS
SC_BATCH = 12288
B_PER_W = SC_BATCH // NW
L_ROWS = B_PER_W // 128
RING = 16
N_GROUPS = B_PER_W // RING

TC_BATCH = BATCH - SC_BATCH
TC_PER_STEP = 8
TC_STEPS = TC_BATCH // TC_PER_STEP

_mesh = plsc.VectorSubcoreMesh(core_axis_name="c", subcore_axis_name="s")


@functools.partial(
    pl.kernel,
    mesh=_mesh,
    compiler_params=pltpu.CompilerParams(needs_layout_passes=False),
    out_type=jax.ShapeDtypeStruct((NW, 8, 128), jnp.float32),
    scratch_types=[
        pltpu.VMEM((L_ROWS, 128), jnp.int32),
        pltpu.VMEM((FEAT_DIM, B_PER_W), jnp.float32),
        pltpu.VMEM((RING, FEAT_DIM, 128), jnp.float32),
        pltpu.VMEM((8, 128), jnp.float32),
        [pltpu.SemaphoreType.DMA] * RING,
    ],
)
def _center_loss_sc(xt_hbm, labels_hbm, ct_hbm, out_hbm,
                    idx_v, x_v, blk_v, acc_v, sems):
    wid = lax.axis_index("s") * NC + lax.axis_index("c")
    base = pl.multiple_of(wid * B_PER_W, 128)

    pltpu.sync_copy(labels_hbm.at[wid], idx_v)
    pltpu.sync_copy(xt_hbm.at[:, pl.ds(base, B_PER_W)], x_v)

    iota16 = lax.iota(jnp.int32, 16)
    f_lo = iota16
    f_hi = iota16 + 16
    zeros16 = jnp.zeros((16,), jnp.float32)

    def group_labels(g):
        return idx_v[g // 8, pl.ds((g % 8) * 16, 16)]

    def fire(tv, k):
        t = pl.multiple_of((tv[k] >> 7) << 7, 128)
        for i in range(4):
            pltpu.async_copy(ct_hbm.at[pl.ds(i * 8, 8), pl.ds(t, 128)],
                             blk_v.at[k, pl.ds(i * 8, 8)], sems[k])

    lv0 = group_labels(0)
    for k in range(RING):
        fire(lv0, k)

    def group(g, carry):
        acc, lv = carry
        rem = lax.rem(lv, 128)
        lv_next = group_labels(jnp.minimum(g + 1, N_GROUPS - 1))
        for k in range(RING):
            for i in range(4):
                pltpu.make_async_copy(
                    ct_hbm.at[pl.ds(0, 8), pl.ds(0, 128)],
                    blk_v.at[k, pl.ds(0, 8)], sems[k]).wait()
            lane16 = jnp.full((16,), rem[k], jnp.int32)
            k16 = jnp.full((16,), k, jnp.int32)
            i16 = jnp.full((16,), g * RING + k, jnp.int32)
            c_lo = plsc.load_gather(blk_v, [k16, f_lo, lane16])
            c_hi = plsc.load_gather(blk_v, [k16, f_hi, lane16])
            x_lo = plsc.load_gather(x_v, [f_lo, i16])
            x_hi = plsc.load_gather(x_v, [f_hi, i16])
            d_lo = x_lo - c_lo
            d_hi = x_hi - c_hi
            acc = acc + d_lo * d_lo + d_hi * d_hi

            @pl.when(g < N_GROUPS - 1)
            def _():
                fire(lv_next, k)
        return acc, lv_next

    acc, _ = lax.fori_loop(
        0, N_GROUPS, group, (jnp.zeros((16,), jnp.float32), lv0))

    for r in range(8):
        for c in range(0, 128, 16):
            if r == 0 and c == 0:
                continue
            acc_v[r, pl.ds(c, 16)] = zeros16
    acc_v[0, pl.ds(0, 16)] = acc
    pltpu.sync_copy(acc_v, out_hbm.at[wid])


def _tc_body(lab_ref, *refs):
    xblk_ref = refs[0]
    cblks = refs[1:1 + TC_PER_STEP]
    out_ref = refs[1 + TC_PER_STEP]
    pid = pl.program_id(0)

    @pl.when(pid == 0)
    def _():
        out_ref[...] = jnp.zeros((1, 1), jnp.float32)

    lanes = lax.broadcasted_iota(jnp.int32, (FEAT_DIM, 128), 1)
    s = jnp.float32(0.0)
    for j in range(TC_PER_STEP):
        i = pid * TC_PER_STEP + j
        L = lab_ref[i]
        rem = lax.rem(L, 128)
        xcol = lax.rem(SC_BATCH + i, 128)
        c_sel = jnp.where(lanes == rem, cblks[j][...], 0.0)
        x_sel = jnp.where(lanes == xcol, xblk_ref[...], 0.0)
        d = jnp.sum(x_sel - c_sel, axis=1)
        s = s + jnp.sum(d * d)
    out_ref[...] += jnp.reshape(s, (1, 1))


def _center_loss_tc(xt, labels_tc, ct):
    grid_spec = pltpu.PrefetchScalarGridSpec(
        num_scalar_prefetch=1,
        grid=(TC_STEPS,),
        in_specs=[
            pl.BlockSpec(
                (FEAT_DIM, 128),
                lambda i, lab: (0, (SC_BATCH + i * TC_PER_STEP) // 128)),
        ] + [
            pl.BlockSpec(
                (FEAT_DIM, 128),
                functools.partial(
                    lambda j, i, lab: (0, lab[i * TC_PER_STEP + j] // 128), j))
            for j in range(TC_PER_STEP)
        ],
        out_specs=pl.BlockSpec((1, 1), lambda i, lab: (0, 0)),
    )
    return pl.pallas_call(
        _tc_body,
        grid_spec=grid_spec,
        out_shape=jax.ShapeDtypeStruct((1, 1), jnp.float32),
        compiler_params=pltpu.CompilerParams(
            dimension_semantics=("arbitrary",)),
    )(labels_tc, xt, *([ct] * TC_PER_STEP))


def kernel(x, labels, centers):
    labels_i = labels.astype(jnp.int32)
    labels3 = labels_i[:SC_BATCH].reshape(NW, L_ROWS, 128)
    xt = x.T
    ct = centers.T
    sc_part = _center_loss_sc(xt, labels3, ct)
    tc_part = _center_loss_tc(xt, labels_i[SC_BATCH:], ct)
    return jnp.sum(sc_part) + tc_part[0, 0]


# speedup vs baseline: 2.9788x; 2.9788x over previous
import functools

import jax
import jax.numpy as jnp
from jax import lax
from jax.experimental import pallas as pl
from jax.experimental.pallas import tpu as pltpu
from jax.experimental.pallas import tpu_sc as plsc

NUM_CLASSES = 1000000
FEAT_DIM = 32
BATCH = 16384

NC = 2
NS = 16
NW = NC * NS
B_PER_W = BATCH // NW
RING = 16
N_GROUPS = B_PER_W // RING

_mesh = plsc.VectorSubcoreMesh(core_axis_name="c", subcore_axis_name="s")


@functools.partial(
    pl.kernel,
    mesh=_mesh,
    compiler_params=pltpu.CompilerParams(needs_layout_passes=False),
    out_type=jax.ShapeDtypeStruct((NW, 8, 128), jnp.float32),
    scratch_types=[
        pltpu.VMEM((4, 128), jnp.int32),
        pltpu.VMEM((FEAT_DIM, B_PER_W), jnp.float32),
        pltpu.VMEM((RING, FEAT_DIM, 128), jnp.float32),
        pltpu.VMEM((8, 128), jnp.float32),
        [pltpu.SemaphoreType.DMA] * RING,
    ],
)
def _center_loss_kernel(xt_hbm, labels_hbm, ct_hbm, out_hbm,
                        idx_v, x_v, blk_v, acc_v, sems):
    wid = lax.axis_index("s") * NC + lax.axis_index("c")
    base = pl.multiple_of(wid * B_PER_W, 128)

    pltpu.sync_copy(labels_hbm.at[wid], idx_v)
    pltpu.sync_copy(xt_hbm.at[:, pl.ds(base, B_PER_W)], x_v)

    iota16 = lax.iota(jnp.int32, 16)
    f_lo = iota16
    f_hi = iota16 + 16
    zeros16 = jnp.zeros((16,), jnp.float32)

    def group_labels(g):
        return idx_v[g // 8, pl.ds((g % 8) * 16, 16)]

    def fire(tv, k):
        t = pl.multiple_of((tv[k] >> 7) << 7, 128)
        for i in range(4):
            pltpu.async_copy(ct_hbm.at[pl.ds(i * 8, 8), pl.ds(t, 128)],
                             blk_v.at[k, pl.ds(i * 8, 8)], sems[k])

    lv0 = group_labels(0)
    for k in range(RING):
        fire(lv0, k)

    def group(g, carry):
        acc, lv = carry
        rem = lax.rem(lv, 128)
        lv_next = group_labels(jnp.minimum(g + 1, N_GROUPS - 1))
        for k in range(RING):
            for i in range(4):
                pltpu.make_async_copy(
                    ct_hbm.at[pl.ds(0, 8), pl.ds(0, 128)],
                    blk_v.at[k, pl.ds(0, 8)], sems[k]).wait()
            lane16 = jnp.full((16,), rem[k], jnp.int32)
            k16 = jnp.full((16,), k, jnp.int32)
            i16 = jnp.full((16,), g * RING + k, jnp.int32)
            c_lo = plsc.load_gather(blk_v, [k16, f_lo, lane16])
            c_hi = plsc.load_gather(blk_v, [k16, f_hi, lane16])
            x_lo = plsc.load_gather(x_v, [f_lo, i16])
            x_hi = plsc.load_gather(x_v, [f_hi, i16])
            d_lo = x_lo - c_lo
            d_hi = x_hi - c_hi
            acc = acc + d_lo * d_lo + d_hi * d_hi

            @pl.when(g < N_GROUPS - 1)
            def _():
                fire(lv_next, k)
        return acc, lv_next

    acc, _ = lax.fori_loop(
        0, N_GROUPS, group, (jnp.zeros((16,), jnp.float32), lv0))

    for r in range(8):
        for c in range(0, 128, 16):
            if r == 0 and c == 0:
                continue
            acc_v[r, pl.ds(c, 16)] = zeros16
    acc_v[0, pl.ds(0, 16)] = acc
    pltpu.sync_copy(acc_v, out_hbm.at[wid])


def kernel(x, labels, centers):
    labels3 = labels.astype(jnp.int32).reshape(NW, 4, 128)
    partials = _center_loss_kernel(x.T, labels3, centers.T)
    return jnp.sum(partials)
